# baseline (speedup 1.0000x reference)
import jax
import jax.numpy as jnp
from jax import lax
from jax.experimental import pallas as pl
from jax.experimental.pallas import tpu as pltpu
from jax.experimental.pallas import tpu_sc as plsc

_BINWIDTHS = (100, 200, 500, 1000, 2000, 5000)
_CUMSTARTS = (0, 2001, 3002, 3403, 3604, 3705)
_NBINS_TOTAL = 3746
_N_REGIONS = 64
_N_EMB = 100
_N_FRAG = 32768
_N_SLOT = 16

_N_TILES = 16
_FRAGS_PER_TILE = _N_FRAG // _N_TILES
_CHUNK = 16
_N_CHUNK = _FRAGS_PER_TILE // _CHUNK
_ROWS_PER_CHUNK = 24 * _CHUNK

_ROW_PAD = 104

_OFFS = (0, 16, 32, 48, 64, 80, 84)


def _body(w2, coords_t, ipad, regs, out, coords0_vm, coords1_vm, ip_vm,
          reg_vm, idx_a, idx_b, alpha_a, alpha_b, rows_a, rows_b, ostage_a,
          ostage_b, sem_a, sem_b, sem_oa, sem_ob):
    wid = lax.axis_index("s")
    tb = wid * _FRAGS_PER_TILE

    pltpu.sync_copy(coords_t.at[0, pl.ds(tb, _FRAGS_PER_TILE)], coords0_vm)
    pltpu.sync_copy(coords_t.at[1, pl.ds(tb, _FRAGS_PER_TILE)], coords1_vm)
    pltpu.sync_copy(ipad, ip_vm)
    pltpu.sync_copy(regs, reg_vm)

    iota = lax.iota(jnp.int32, 16)

    def phase1(n, idx_vm, alpha_vm):
        fv = tb + n * _CHUNK + iota
        s = jnp.zeros((16,), jnp.int32)
        for j in range(1, 17):
            bj = plsc.load_gather(ip_vm, [jnp.full((16,), j, jnp.int32)])
            s = s + (fv >= bj).astype(jnp.int32)
        slot = jnp.clip(s, 0, _N_SLOT - 1)
        reg = plsc.load_gather(reg_vm, [slot])
        base = reg * _NBINS_TOTAL
        for c in range(2):
            cvm = coords0_vm if c == 0 else coords1_vm
            cv = cvm[pl.ds(n * _CHUNK, _CHUNK)]
            u = jnp.clip(cv, -100000, 99999) + 100000
            for b in range(6):
                bw = _BINWIDTHS[b]
                k = b * 2 + c
                q = lax.div(u, jnp.int32(bw))
                cum0 = jnp.clip(q + _CUMSTARTS[b], 0, _NBINS_TOTAL - 1)
                cum1 = jnp.minimum(cum0 + 1, _NBINS_TOTAL - 1)
                idx_vm[pl.ds(32 * k, 16)] = base + cum0
                idx_vm[pl.ds(32 * k + 16, 16)] = base + cum1
                rem = u - q * jnp.int32(bw)
                alpha_vm[pl.ds(k * 16, 16)] = (
                    rem.astype(jnp.float32) * jnp.float32(1.0 / bw))

    def issue(idx_vm, rows_vm, sem):
        for j in range(3):
            pltpu.async_copy(w2.at[idx_vm.at[pl.ds(j * 128, 128)]],
                             rows_vm.at[pl.ds(j * 128, 128)], sem)

    def drain(idx_vm, rows_vm, sem):
        for j in range(3):
            pltpu.make_async_copy(w2.at[idx_vm.at[pl.ds(j * 128, 128)]],
                                  rows_vm.at[pl.ds(j * 128, 128)],
                                  sem).wait()

    def out_ref(n):
        return out.at[pl.ds((tb + n * _CHUNK) * _N_EMB, _CHUNK * _N_EMB)]

    def phase3(n, alpha_vm, rows_vm, ostage_vm, osem):
        def fbody(f, carry):
            af = plsc.load_gather(alpha_vm, [f + 16 * iota])
            accs = None
            for k in range(12):
                a = af.at[jnp.full((16,), k, jnp.int32)].get(
                    mode="promise_in_bounds")
                r0 = 32 * k + f
                r1 = r0 + 16
                news = []
                for v, off in enumerate(_OFFS):
                    w0 = rows_vm[r0, pl.ds(off, 16)]
                    w1 = rows_vm[r1, pl.ds(off, 16)]
                    t = (w1 - w0) * a + w0
                    news.append(t if accs is None else accs[v] + t)
                accs = news
            for v, off in enumerate(_OFFS):
                ostage_vm[pl.ds(f * _N_EMB + off, 16)] = accs[v]
            return carry

        lax.fori_loop(0, _CHUNK, fbody, 0, unroll=False)
        pltpu.async_copy(ostage_vm, out_ref(n), osem)

    def odrain(n, ostage_vm, osem):
        pltpu.make_async_copy(ostage_vm, out_ref(n), osem).wait()

    phase1(0, idx_a, alpha_a)
    issue(idx_a, rows_a, sem_a)

    def pair(i, carry):
        n0 = 2 * i
        phase1(n0 + 1, idx_b, alpha_b)
        issue(idx_b, rows_b, sem_b)
        drain(idx_a, rows_a, sem_a)

        @pl.when(i > 0)
        def _():
            odrain(n0 - 2, ostage_a, sem_oa)

        phase3(n0, alpha_a, rows_a, ostage_a, sem_oa)

        @pl.when(i < _N_CHUNK // 2 - 1)
        def _():
            phase1(n0 + 2, idx_a, alpha_a)
            issue(idx_a, rows_a, sem_a)

        drain(idx_b, rows_b, sem_b)

        @pl.when(i > 0)
        def _():
            odrain(n0 - 1, ostage_b, sem_ob)

        phase3(n0 + 1, alpha_b, rows_b, ostage_b, sem_ob)
        return carry

    lax.fori_loop(0, _N_CHUNK // 2, pair, 0, unroll=False)
    odrain(_N_CHUNK - 2, ostage_a, sem_oa)
    odrain(_N_CHUNK - 1, ostage_b, sem_ob)


def kernel(coordinates, indptr, regions_oi, W):
    coords_t = coordinates.T
    w2 = jnp.pad(W.reshape(_N_REGIONS * _NBINS_TOTAL, _N_EMB),
                 ((0, 0), (0, _ROW_PAD - _N_EMB)))
    ipad = jnp.concatenate(
        [indptr.astype(jnp.int32), jnp.zeros((7,), jnp.int32)])
    mesh = plsc.VectorSubcoreMesh(core_axis_name="c", subcore_axis_name="s",
                                  num_cores=1, num_subcores=16)
    run = pl.kernel(
        _body,
        out_type=jax.ShapeDtypeStruct((_N_FRAG * _N_EMB,), jnp.float32),
        mesh=mesh,
        compiler_params=pltpu.CompilerParams(needs_layout_passes=False,
                                             use_tc_tiling_on_sc=False),
        scratch_types=[
            pltpu.VMEM((_FRAGS_PER_TILE,), jnp.int32),
            pltpu.VMEM((_FRAGS_PER_TILE,), jnp.int32),
            pltpu.VMEM((24,), jnp.int32),
            pltpu.VMEM((16,), jnp.int32),
            pltpu.VMEM((_ROWS_PER_CHUNK,), jnp.int32),
            pltpu.VMEM((_ROWS_PER_CHUNK,), jnp.int32),
            pltpu.VMEM((256,), jnp.float32),
            pltpu.VMEM((256,), jnp.float32),
            pltpu.VMEM((_ROWS_PER_CHUNK, _ROW_PAD), jnp.float32),
            pltpu.VMEM((_ROWS_PER_CHUNK, _ROW_PAD), jnp.float32),
            pltpu.VMEM((_CHUNK * _N_EMB,), jnp.float32),
            pltpu.VMEM((_CHUNK * _N_EMB,), jnp.float32),
            pltpu.SemaphoreType.DMA,
            pltpu.SemaphoreType.DMA,
            pltpu.SemaphoreType.DMA,
            pltpu.SemaphoreType.DMA,
        ],
    )
    out = run(w2, coords_t, ipad, regions_oi)
    return out.reshape(_N_FRAG, _N_EMB)


# speedup vs baseline: 1.1574x; 1.1574x over previous
import jax
import jax.numpy as jnp
from jax import lax
from jax.experimental import pallas as pl
from jax.experimental.pallas import tpu as pltpu
from jax.experimental.pallas import tpu_sc as plsc

_BINWIDTHS = (100, 200, 500, 1000, 2000, 5000)
_CUMSTARTS = (0, 2001, 3002, 3403, 3604, 3705)
_NBINS_TOTAL = 3746
_N_REGIONS = 64
_N_EMB = 100
_N_FRAG = 32768
_N_SLOT = 16

_N_TILES = 32
_FRAGS_PER_TILE = _N_FRAG // _N_TILES
_CHUNK = 16
_N_CHUNK = _FRAGS_PER_TILE // _CHUNK
_ROWS_PER_CHUNK = 24 * _CHUNK

_ROW_PAD = 104

_OFFS = (0, 16, 32, 48, 64, 80, 84)


def _body(w2, coords_t, ipad, regs, out, coords0_vm, coords1_vm, ip_vm,
          reg_vm, idx_a, idx_b, alpha_a, alpha_b, rows_a, rows_b, ostage_a,
          ostage_b, sem_a, sem_b, sem_oa, sem_ob):
    wid = lax.axis_index("s") * 2 + lax.axis_index("c")
    tb = wid * _FRAGS_PER_TILE

    pltpu.sync_copy(coords_t.at[0, pl.ds(tb, _FRAGS_PER_TILE)], coords0_vm)
    pltpu.sync_copy(coords_t.at[1, pl.ds(tb, _FRAGS_PER_TILE)], coords1_vm)
    pltpu.sync_copy(ipad, ip_vm)
    pltpu.sync_copy(regs, reg_vm)

    iota = lax.iota(jnp.int32, 16)

    def phase1(n, idx_vm, alpha_vm):
        fv = tb + n * _CHUNK + iota
        s = jnp.zeros((16,), jnp.int32)
        for j in range(1, 17):
            bj = plsc.load_gather(ip_vm, [jnp.full((16,), j, jnp.int32)])
            s = s + (fv >= bj).astype(jnp.int32)
        slot = jnp.clip(s, 0, _N_SLOT - 1)
        reg = plsc.load_gather(reg_vm, [slot])
        base = reg * _NBINS_TOTAL
        for c in range(2):
            cvm = coords0_vm if c == 0 else coords1_vm
            cv = cvm[pl.ds(n * _CHUNK, _CHUNK)]
            u = jnp.clip(cv, -100000, 99999) + 100000
            for b in range(6):
                bw = _BINWIDTHS[b]
                k = b * 2 + c
                q = lax.div(u, jnp.int32(bw))
                cum0 = jnp.clip(q + _CUMSTARTS[b], 0, _NBINS_TOTAL - 1)
                cum1 = jnp.minimum(cum0 + 1, _NBINS_TOTAL - 1)
                idx_vm[pl.ds(32 * k, 16)] = base + cum0
                idx_vm[pl.ds(32 * k + 16, 16)] = base + cum1
                rem = u - q * jnp.int32(bw)
                alpha_vm[pl.ds(k * 16, 16)] = (
                    rem.astype(jnp.float32) * jnp.float32(1.0 / bw))

    def issue(idx_vm, rows_vm, sem):
        for j in range(3):
            pltpu.async_copy(w2.at[idx_vm.at[pl.ds(j * 128, 128)]],
                             rows_vm.at[pl.ds(j * 128, 128)], sem)

    def drain(idx_vm, rows_vm, sem):
        for j in range(3):
            pltpu.make_async_copy(w2.at[idx_vm.at[pl.ds(j * 128, 128)]],
                                  rows_vm.at[pl.ds(j * 128, 128)],
                                  sem).wait()

    def out_ref(n):
        return out.at[pl.ds((tb + n * _CHUNK) * _N_EMB, _CHUNK * _N_EMB)]

    def phase3(n, alpha_vm, rows_vm, ostage_vm, osem):
        def fbody(f, carry):
            af = plsc.load_gather(alpha_vm, [f + 16 * iota])
            accs = None
            for k in range(12):
                a = af.at[jnp.full((16,), k, jnp.int32)].get(
                    mode="promise_in_bounds")
                r0 = 32 * k + f
                r1 = r0 + 16
                news = []
                for v, off in enumerate(_OFFS):
                    w0 = rows_vm[r0, pl.ds(off, 16)]
                    w1 = rows_vm[r1, pl.ds(off, 16)]
                    t = (w1 - w0) * a + w0
                    news.append(t if accs is None else accs[v] + t)
                accs = news
            for v, off in enumerate(_OFFS):
                ostage_vm[pl.ds(f * _N_EMB + off, 16)] = accs[v]
            return carry

        lax.fori_loop(0, _CHUNK, fbody, 0, unroll=False)
        pltpu.async_copy(ostage_vm, out_ref(n), osem)

    def odrain(n, ostage_vm, osem):
        pltpu.make_async_copy(ostage_vm, out_ref(n), osem).wait()

    phase1(0, idx_a, alpha_a)
    issue(idx_a, rows_a, sem_a)

    def pair(i, carry):
        n0 = 2 * i
        phase1(n0 + 1, idx_b, alpha_b)
        issue(idx_b, rows_b, sem_b)
        drain(idx_a, rows_a, sem_a)

        @pl.when(i > 0)
        def _():
            odrain(n0 - 2, ostage_a, sem_oa)

        phase3(n0, alpha_a, rows_a, ostage_a, sem_oa)

        @pl.when(i < _N_CHUNK // 2 - 1)
        def _():
            phase1(n0 + 2, idx_a, alpha_a)
            issue(idx_a, rows_a, sem_a)

        drain(idx_b, rows_b, sem_b)

        @pl.when(i > 0)
        def _():
            odrain(n0 - 1, ostage_b, sem_ob)

        phase3(n0 + 1, alpha_b, rows_b, ostage_b, sem_ob)
        return carry

    lax.fori_loop(0, _N_CHUNK // 2, pair, 0, unroll=False)
    odrain(_N_CHUNK - 2, ostage_a, sem_oa)
    odrain(_N_CHUNK - 1, ostage_b, sem_ob)


def kernel(coordinates, indptr, regions_oi, W):
    coords_t = coordinates.T
    w2 = jnp.pad(W.reshape(_N_REGIONS * _NBINS_TOTAL, _N_EMB),
                 ((0, 0), (0, _ROW_PAD - _N_EMB)))
    ipad = jnp.concatenate(
        [indptr.astype(jnp.int32), jnp.zeros((7,), jnp.int32)])
    mesh = plsc.VectorSubcoreMesh(core_axis_name="c", subcore_axis_name="s",
                                  num_cores=2, num_subcores=16)
    run = pl.kernel(
        _body,
        out_type=jax.ShapeDtypeStruct((_N_FRAG * _N_EMB,), jnp.float32),
        mesh=mesh,
        compiler_params=pltpu.CompilerParams(needs_layout_passes=False,
                                             use_tc_tiling_on_sc=False),
        scratch_types=[
            pltpu.VMEM((_FRAGS_PER_TILE,), jnp.int32),
            pltpu.VMEM((_FRAGS_PER_TILE,), jnp.int32),
            pltpu.VMEM((24,), jnp.int32),
            pltpu.VMEM((16,), jnp.int32),
            pltpu.VMEM((_ROWS_PER_CHUNK,), jnp.int32),
            pltpu.VMEM((_ROWS_PER_CHUNK,), jnp.int32),
            pltpu.VMEM((256,), jnp.float32),
            pltpu.VMEM((256,), jnp.float32),
            pltpu.VMEM((_ROWS_PER_CHUNK, _ROW_PAD), jnp.float32),
            pltpu.VMEM((_ROWS_PER_CHUNK, _ROW_PAD), jnp.float32),
            pltpu.VMEM((_CHUNK * _N_EMB,), jnp.float32),
            pltpu.VMEM((_CHUNK * _N_EMB,), jnp.float32),
            pltpu.SemaphoreType.DMA,
            pltpu.SemaphoreType.DMA,
            pltpu.SemaphoreType.DMA,
            pltpu.SemaphoreType.DMA,
        ],
    )
    out = run(w2, coords_t, ipad, regions_oi)
    return out.reshape(_N_FRAG, _N_EMB)


# speedup vs baseline: 1.3826x; 1.1946x over previous
import jax
import jax.numpy as jnp
from jax import lax
from jax.experimental import pallas as pl
from jax.experimental.pallas import tpu as pltpu
from jax.experimental.pallas import tpu_sc as plsc

_BINWIDTHS = (100, 200, 500, 1000, 2000, 5000)
_CUMSTARTS = (0, 2001, 3002, 3403, 3604, 3705)
_NBINS_TOTAL = 3746
_N_REGIONS = 64
_N_EMB = 100
_N_FRAG = 32768
_N_SLOT = 16

_N_TILES = 32
_FRAGS_PER_TILE = _N_FRAG // _N_TILES
_CHUNK = 16
_N_CHUNK = _FRAGS_PER_TILE // _CHUNK
_ROWS_PER_CHUNK = 24 * _CHUNK

_ROW_PAD = 128

_OFFS = (0, 16, 32, 48, 64, 80, 84)


def _body(w2, coords_t, ipad, regs, out, coords0_vm, coords1_vm, ip_vm,
          reg_vm, idx_a, idx_b, alpha_a, alpha_b, rows_a, rows_b, ostage_a,
          ostage_b, sem_a, sem_b, sem_oa, sem_ob):
    wid = lax.axis_index("s") * 2 + lax.axis_index("c")
    tb = wid * _FRAGS_PER_TILE

    pltpu.sync_copy(coords_t.at[0, pl.ds(tb, _FRAGS_PER_TILE)], coords0_vm)
    pltpu.sync_copy(coords_t.at[1, pl.ds(tb, _FRAGS_PER_TILE)], coords1_vm)
    pltpu.sync_copy(ipad, ip_vm)
    pltpu.sync_copy(regs, reg_vm)

    iota = lax.iota(jnp.int32, 16)

    def phase1(n, idx_vm, alpha_vm):
        fv = tb + n * _CHUNK + iota
        s = jnp.zeros((16,), jnp.int32)
        for j in range(1, 17):
            bj = plsc.load_gather(ip_vm, [jnp.full((16,), j, jnp.int32)])
            s = s + (fv >= bj).astype(jnp.int32)
        slot = jnp.clip(s, 0, _N_SLOT - 1)
        reg = plsc.load_gather(reg_vm, [slot])
        base = reg * _NBINS_TOTAL
        for c in range(2):
            cvm = coords0_vm if c == 0 else coords1_vm
            cv = cvm[pl.ds(n * _CHUNK, _CHUNK)]
            u = jnp.clip(cv, -100000, 99999) + 100000
            for b in range(6):
                bw = _BINWIDTHS[b]
                k = b * 2 + c
                q = lax.div(u, jnp.int32(bw))
                cum0 = jnp.clip(q + _CUMSTARTS[b], 0, _NBINS_TOTAL - 1)
                cum1 = jnp.minimum(cum0 + 1, _NBINS_TOTAL - 1)
                idx_vm[pl.ds(32 * k, 16)] = base + cum0
                idx_vm[pl.ds(32 * k + 16, 16)] = base + cum1
                rem = u - q * jnp.int32(bw)
                alpha_vm[pl.ds(k * 16, 16)] = (
                    rem.astype(jnp.float32) * jnp.float32(1.0 / bw))

    def issue(idx_vm, rows_vm, sem):
        for j in range(3):
            pltpu.async_copy(w2.at[idx_vm.at[pl.ds(j * 128, 128)]],
                             rows_vm.at[pl.ds(j * 128, 128)], sem)

    def drain(idx_vm, rows_vm, sem):
        for j in range(3):
            pltpu.make_async_copy(w2.at[idx_vm.at[pl.ds(j * 128, 128)]],
                                  rows_vm.at[pl.ds(j * 128, 128)],
                                  sem).wait()

    def out_ref(n):
        return out.at[pl.ds((tb + n * _CHUNK) * _N_EMB, _CHUNK * _N_EMB)]

    def phase3(n, alpha_vm, rows_vm, ostage_vm, osem):
        def fbody(f, carry):
            af = plsc.load_gather(alpha_vm, [f + 16 * iota])
            accs = None
            for k in range(12):
                a = af.at[jnp.full((16,), k, jnp.int32)].get(
                    mode="promise_in_bounds")
                r0 = 32 * k + f
                r1 = r0 + 16
                news = []
                for v, off in enumerate(_OFFS):
                    w0 = rows_vm[r0, pl.ds(off, 16)]
                    w1 = rows_vm[r1, pl.ds(off, 16)]
                    t = (w1 - w0) * a + w0
                    news.append(t if accs is None else accs[v] + t)
                accs = news
            for v, off in enumerate(_OFFS):
                ostage_vm[pl.ds(f * _N_EMB + off, 16)] = accs[v]
            return carry

        lax.fori_loop(0, _CHUNK, fbody, 0, unroll=False)
        pltpu.async_copy(ostage_vm, out_ref(n), osem)

    def odrain(n, ostage_vm, osem):
        pltpu.make_async_copy(ostage_vm, out_ref(n), osem).wait()

    phase1(0, idx_a, alpha_a)
    issue(idx_a, rows_a, sem_a)

    def pair(i, carry):
        n0 = 2 * i
        phase1(n0 + 1, idx_b, alpha_b)
        issue(idx_b, rows_b, sem_b)
        drain(idx_a, rows_a, sem_a)

        @pl.when(i > 0)
        def _():
            odrain(n0 - 2, ostage_a, sem_oa)

        phase3(n0, alpha_a, rows_a, ostage_a, sem_oa)

        @pl.when(i < _N_CHUNK // 2 - 1)
        def _():
            phase1(n0 + 2, idx_a, alpha_a)
            issue(idx_a, rows_a, sem_a)

        drain(idx_b, rows_b, sem_b)

        @pl.when(i > 0)
        def _():
            odrain(n0 - 1, ostage_b, sem_ob)

        phase3(n0 + 1, alpha_b, rows_b, ostage_b, sem_ob)
        return carry

    lax.fori_loop(0, _N_CHUNK // 2, pair, 0, unroll=False)
    odrain(_N_CHUNK - 2, ostage_a, sem_oa)
    odrain(_N_CHUNK - 1, ostage_b, sem_ob)


def kernel(coordinates, indptr, regions_oi, W):
    coords_t = coordinates.T
    w2 = jnp.pad(W.reshape(_N_REGIONS * _NBINS_TOTAL, _N_EMB),
                 ((0, 0), (0, _ROW_PAD - _N_EMB)))
    ipad = jnp.concatenate(
        [indptr.astype(jnp.int32), jnp.zeros((7,), jnp.int32)])
    mesh = plsc.VectorSubcoreMesh(core_axis_name="c", subcore_axis_name="s",
                                  num_cores=2, num_subcores=16)
    run = pl.kernel(
        _body,
        out_type=jax.ShapeDtypeStruct((_N_FRAG * _N_EMB,), jnp.float32),
        mesh=mesh,
        compiler_params=pltpu.CompilerParams(needs_layout_passes=False,
                                             use_tc_tiling_on_sc=True),
        scratch_types=[
            pltpu.VMEM((_FRAGS_PER_TILE,), jnp.int32),
            pltpu.VMEM((_FRAGS_PER_TILE,), jnp.int32),
            pltpu.VMEM((24,), jnp.int32),
            pltpu.VMEM((16,), jnp.int32),
            pltpu.VMEM((_ROWS_PER_CHUNK,), jnp.int32),
            pltpu.VMEM((_ROWS_PER_CHUNK,), jnp.int32),
            pltpu.VMEM((256,), jnp.float32),
            pltpu.VMEM((256,), jnp.float32),
            pltpu.VMEM((_ROWS_PER_CHUNK, _ROW_PAD), jnp.float32),
            pltpu.VMEM((_ROWS_PER_CHUNK, _ROW_PAD), jnp.float32),
            pltpu.VMEM((_CHUNK * _N_EMB,), jnp.float32),
            pltpu.VMEM((_CHUNK * _N_EMB,), jnp.float32),
            pltpu.SemaphoreType.DMA,
            pltpu.SemaphoreType.DMA,
            pltpu.SemaphoreType.DMA,
            pltpu.SemaphoreType.DMA,
        ],
    )
    out = run(w2, coords_t, ipad, regions_oi)
    return out.reshape(_N_FRAG, _N_EMB)
